# baseline (speedup 1.0000x reference)
import functools

import jax
import jax.numpy as jnp
from jax import lax
from jax.experimental import pallas as pl
from jax.experimental.pallas import tpu as pltpu
from jax.experimental.pallas import tpu_sc as plsc

_BATCH = 16384
_SEQ = 200
_DIM = 32
_VOCAB = 1000
_VPAD = 1024
_NUM_CORES = 2
_NUM_SUBCORES = 16
_NW = _NUM_CORES * _NUM_SUBCORES
_BT = _BATCH // 128
_BT_PER_W = _BT // _NW
_BW = _BT_PER_W * 128
_GROUPS = _BW // 16
_PIECE = 8 * 512
_PLANE = _DIM * _BATCH
_LANES = 16

_mesh = plsc.VectorSubcoreMesh(core_axis_name="c", subcore_axis_name="s")


@functools.partial(
    pl.kernel,
    mesh=_mesh,
    out_type=jax.ShapeDtypeStruct((_BATCH * _SEQ * _DIM,), jnp.float32),
    scratch_types=[
        pltpu.VMEM((_BW,), jnp.int32),
        pltpu.VMEM((_BW,), jnp.int32),
        pltpu.VMEM((4 * _PIECE,), jnp.float32),
        pltpu.VMEM((4 * _PIECE,), jnp.float32),
        pltpu.VMEM((4 * (_DIM // 2) * _VPAD,), jnp.int32),
        pltpu.SemaphoreType.DMA,
        pltpu.SemaphoreType.DMA,
        pltpu.SemaphoreType.DMA,
        pltpu.SemaphoreType.DMA,
    ],
    compiler_params=pltpu.CompilerParams(use_tc_tiling_on_sc=False,
                                         needs_layout_passes=False),
)
def _gather_kernel(img_hbm, tabp_hbm, out_hbm,
                   idx0, idx1, buf0, buf1, table_v,
                   sl0, sl1, ss0, ss1):
    sid = lax.axis_index("s")
    tid = sid * _NUM_CORES + lax.axis_index("c")
    bt0 = tid * _BT_PER_W

    idx = (idx0, idx1)
    buf = (buf0, buf1)
    sl = (sl0, sl1)
    ss = (ss0, ss1)

    pltpu.sync_copy(tabp_hbm, table_v)
    lane_r = lax.iota(jnp.int32, _LANES) & 3

    def issue_l(s, p):
        st = s // 8
        s8 = s % 8
        for k in range(_BT_PER_W):
            pltpu.async_copy(img_hbm.at[st, bt0 + k, s8],
                             idx[p].at[pl.ds(k * 128, 128)], sl[p])

    def wait_l(p):
        for k in range(_BT_PER_W):
            pltpu.make_async_copy(img_hbm.at[0, 0, 0],
                                  idx[p].at[pl.ds(k * 128, 128)],
                                  sl[p]).wait()

    def issue_s(s, p):
        for dt in range(4):
            pltpu.async_copy(
                buf[p].at[pl.ds(dt * _PIECE, _PIECE)],
                out_hbm.at[pl.ds(s * _PLANE + dt * (8 * _BATCH)
                                 + bt0 * 1024, _PIECE)],
                ss[p])

    def wait_s(p):
        for dt in range(4):
            pltpu.make_async_copy(buf[p].at[pl.ds(dt * _PIECE, _PIECE)],
                                  out_hbm.at[pl.ds(0, _PIECE)],
                                  ss[p]).wait()

    def compute(p):
        idx_ref = idx[p]
        buf_ref = buf[p]

        @plsc.parallel_loop(0, _GROUPS, unroll=4)
        def group(g):
            idx16 = (idx_ref[pl.ds(g * _LANES, _LANES)] << 2) + lane_r
            gbase = (g >> 3) * 1024 + (g & 7) * _LANES
            for j in range(_DIM // 2):
                d = 2 * j
                dt, d8 = divmod(d, 8)
                w = plsc.load_gather(table_v, [idx16 + j * (4 * _VPAD)])
                lo = plsc.bitcast(w << 16, jnp.float32)
                hi = plsc.bitcast(w & jnp.int32(-65536), jnp.float32)
                base = dt * _PIECE + gbase + d8 * 128
                buf_ref[pl.ds(base, _LANES)] = lo
                buf_ref[pl.ds(base + 128, _LANES)] = hi

    issue_l(0, 0)

    def step(s, p, op):
        @pl.when(s >= 2)
        def _():
            wait_s(p)

        wait_l(p)

        @pl.when(s + 1 < _SEQ)
        def _():
            issue_l(s + 1, op)

        compute(p)
        issue_s(s, p)

    def outer(g, carry):
        step(2 * g, 0, 1)
        step(2 * g + 1, 1, 0)
        return carry

    lax.fori_loop(0, _SEQ // 2, outer, 0)

    wait_s(0)
    wait_s(1)


def kernel(sentence, table):
    img = sentence.reshape(_BATCH // 128, 128, _SEQ // 8, 8)
    img = img.transpose(2, 0, 3, 1)
    tb = table.astype(jnp.bfloat16).reshape(_VOCAB, _DIM // 2, 2)
    tw = jax.lax.bitcast_convert_type(tb, jnp.uint16)
    packed = (tw[..., 0].astype(jnp.int32)
              | (tw[..., 1].astype(jnp.int32) << 16))
    packed = jnp.pad(packed, ((0, _VPAD - _VOCAB), (0, 0))).T
    packed = jnp.broadcast_to(packed[:, :, None], (16, _VPAD, 4)).reshape(-1)
    out = _gather_kernel(img, packed)
    out = out.reshape(_SEQ, _DIM // 8, _BATCH // 128, 8, 128)
    out = out.transpose(2, 4, 0, 1, 3).reshape(_BATCH, _SEQ, _DIM)
    return out
